# baseline (speedup 1.0000x reference)
import jax
import jax.numpy as jnp
import numpy as np
from jax import lax
from jax.experimental import pallas as pl
from jax.experimental.pallas import tpu as pltpu
from jax.experimental.pallas import tpu_sc as plsc

N_ROWS = 2097152
N_FEAT = 16
LOG2NB = 15
NB = 1 << LOG2NB
SHIFT = 32 - LOG2NB
HALF = NB // 2
NQ = 256
NC = 2
NS = 16
NW = NC * NS
ROWS_PER_W = N_ROWS // NW
CHUNK = 256
NCHUNKS = ROWS_PER_W // CHUNK
BLK = 16384
NBLK = NB // BLK

_MASK31 = np.int32(0x7FFFFFFF)


def _rank_tables():
    k = np.arange(NQ, dtype=np.float64)
    pos = k * (N_ROWS - 1) / (NQ - 1)
    rlo = np.floor(pos).astype(np.int64)
    frac = (pos - rlo).astype(np.float32)
    rhi = np.minimum(rlo + 1, N_ROWS - 1)
    ranks = np.empty((2 * NQ,), dtype=np.int32)
    ranks[0::2] = rlo
    ranks[1::2] = rhi
    return ranks, frac


def _hist_body(x_hbm, ranks_hbm, hist_hbm, mm_hbm,
               in0, in1, idx0, idx1, ones_buf, mm_buf, mmred_buf,
               hist_sh, mm_sh,
               sem_in0, sem_in1, sem_sc0, sem_sc1):
    del ranks_hbm
    c = lax.axis_index("c")
    s = lax.axis_index("s")
    feat_iota = lax.iota(jnp.int32, 16)
    zeros16 = jnp.zeros((16,), jnp.int32)
    ones16 = jnp.ones((16,), jnp.int32)
    ins = (in0, in1)
    idxs = (idx0, idx1)
    sems_in = (sem_in0, sem_in1)
    sems_sc = (sem_sc0, sem_sc1)

    def fill(i, _):
        idx0[pl.ds(i * 16, 16)] = zeros16
        ones_buf[pl.ds(i * 16, 16)] = ones16
        return 0
    lax.fori_loop(0, CHUNK, fill, 0)

    for j in range(NB // (CHUNK * 16)):
        pltpu.sync_copy(idx0,
                        hist_sh.at[pl.ds(s * NB + j * CHUNK * 16, CHUNK * 16)])
    plsc.subcore_barrier()

    wid = c * NS + s
    row_base = wid * ROWS_PER_W

    pltpu.async_copy(x_hbm.at[pl.ds(row_base, CHUNK)], in0, sem_in0)

    def pair_body(p, carry):
        vmin, vmax = carry
        for par in range(2):
            g = p * 2 + par
            buf = ins[par]
            idx = idxs[par]

            nxt = g + 1

            @pl.when(nxt < NCHUNKS)
            def _():
                pltpu.async_copy(
                    x_hbm.at[pl.ds(row_base + nxt * CHUNK, CHUNK)],
                    ins[1 - par], sems_in[1 - par])

            pltpu.make_async_copy(
                x_hbm.at[pl.ds(row_base, CHUNK)], buf, sems_in[par]).wait()

            @pl.when(p > 0)
            def _():
                pltpu.make_async_copy(
                    ones_buf, hist_sh.at[idx], sems_sc[par]).wait()

            @plsc.parallel_loop(0, CHUNK, unroll=8, carry=(vmin, vmax))
            def mm_carry(i, mc):
                vmn, vmx = mc
                v = buf[i]
                u = plsc.bitcast(v, jnp.int32)
                k = jnp.where(u >= 0, u, u ^ _MASK31)
                b = (k >> SHIFT) + HALF
                idx[pl.ds(i * 16, 16)] = b + feat_iota * NB
                return jnp.minimum(vmn, v), jnp.maximum(vmx, v)

            vmin, vmax = mm_carry
            pltpu.async_copy(ones_buf, hist_sh.at[idx], sems_sc[par], add=True)
        return vmin, vmax

    inf16 = jnp.full((16,), jnp.inf, jnp.float32)
    vmin, vmax = lax.fori_loop(0, NCHUNKS // 2, pair_body, (inf16, -inf16))
    for par in range(2):
        pltpu.make_async_copy(ones_buf, hist_sh.at[idxs[par]],
                              sems_sc[par]).wait()

    mm_buf[pl.ds(0, 16)] = vmin
    mm_buf[pl.ds(16, 16)] = vmax
    pltpu.sync_copy(mm_buf, mm_sh.at[pl.ds(s * 32, 32)])
    plsc.subcore_barrier()

    @pl.when(s == 0)
    def _():
        pltpu.sync_copy(mm_sh, mmred_buf)

        def red(i, mc):
            vmn, vmx = mc
            return (jnp.minimum(vmn, mmred_buf[pl.ds(i * 32, 16)]),
                    jnp.maximum(vmx, mmred_buf[pl.ds(i * 32 + 16, 16)]))
        rmin, rmax = lax.fori_loop(0, NS, red, (inf16, -inf16))
        mm_buf[pl.ds(0, 16)] = rmin
        mm_buf[pl.ds(16, 16)] = rmax
        pltpu.sync_copy(mm_buf, mm_hbm.at[pl.ds(c * 32, 32)])

    for j in range(NB // (CHUNK * 16)):
        pltpu.sync_copy(
            hist_sh.at[pl.ds(s * NB + j * CHUNK * 16, CHUNK * 16)], idx0)
        pltpu.sync_copy(
            idx0,
            hist_hbm.at[pl.ds((c * NS + s) * NB + j * CHUNK * 16,
                              CHUNK * 16)])


def _quant_body(hist_hbm, mm_hbm, ranks_hbm, frac_hbm, out_hbm,
                cum_buf, bufa, bufb, ranks_buf, frac_buf, est_buf,
                q_buf, out_buf, mm_buf):
    c = lax.axis_index("c")
    s = lax.axis_index("s")

    @pl.when(c == 0)
    def _():
        f = s
        iota16 = lax.iota(jnp.int32, 16)

        pltpu.sync_copy(ranks_hbm, ranks_buf)
        pltpu.sync_copy(frac_hbm, frac_buf)
        pltpu.sync_copy(mm_hbm, mm_buf)

        def blk(j, carry):
            pltpu.sync_copy(hist_hbm.at[pl.ds(f * NB + j * BLK, BLK)], bufa)
            pltpu.sync_copy(hist_hbm.at[pl.ds((NS + f) * NB + j * BLK, BLK)],
                            bufb)

            def inner(i, cy):
                va = bufa[pl.ds(i * 16, 16)]
                vb = bufb[pl.ds(i * 16, 16)]
                cs = plsc.cumsum(va + vb) + cy
                cum_buf[pl.ds(j * BLK + i * 16, 16)] = cs
                return jnp.max(cs)
            return lax.fori_loop(0, BLK // 16, inner, carry)

        lax.fori_loop(0, NBLK, blk, jnp.int32(0))

        vmn = jnp.minimum(mm_buf[pl.ds(0, 16)], mm_buf[pl.ds(32, 16)])
        vmx = jnp.maximum(mm_buf[pl.ds(16, 16)], mm_buf[pl.ds(48, 16)])
        minf = jnp.min(jnp.where(iota16 == f, vmn, jnp.inf))
        maxf = jnp.max(jnp.where(iota16 == f, vmx, -jnp.inf))

        for v in range(32):
            r = ranks_buf[pl.ds(v * 16, 16)]
            t = r + 1
            b = jnp.zeros((16,), jnp.int32)
            step = NB // 2
            while step >= 1:
                probe = b + (step - 1)
                val = plsc.load_gather(cum_buf, [probe])
                b = jnp.where(val < t, b + step, b)
                step //= 2
            cb = plsc.load_gather(cum_buf, [b])
            prev = plsc.load_gather(cum_buf, [jnp.maximum(b - 1, 0)])
            cbef = jnp.where(b > 0, prev, 0)
            m = cb - cbef
            j = r - cbef
            fracb = (j.astype(jnp.float32) + 0.5) / m.astype(jnp.float32)
            keylo = (b - HALF) << SHIFT
            keyhi = keylo + ((1 << SHIFT) - 1)
            ulo = jnp.where(keylo >= 0, keylo, keylo ^ _MASK31)
            uhi = jnp.where(keyhi >= 0, keyhi, keyhi ^ _MASK31)
            xlo = plsc.bitcast(ulo, jnp.float32)
            xhi = plsc.bitcast(uhi, jnp.float32)
            est = xlo + fracb * (xhi - xlo)
            est_buf[pl.ds(v * 16, 16)] = est

        for v in range(16):
            idx_even = 32 * v + 2 * iota16
            e_lo = plsc.load_gather(est_buf, [idx_even])
            e_hi = plsc.load_gather(est_buf, [idx_even + 1])
            fr = frac_buf[pl.ds(v * 16, 16)]
            q = e_lo + fr * (e_hi - e_lo)
            if v == 0:
                q = jnp.where(iota16 == 0, minf, q)
            if v == 15:
                q = jnp.where(iota16 == 15, maxf, q)
            q_buf[pl.ds(v * 16, 16)] = q

        for v in range(16):
            out_buf[pl.ds(v * 16, 16)] = jnp.zeros((16,), jnp.float32) + maxf

        pcarry = jnp.int32(0)
        for v in range(16):
            gidx = v * 16 + iota16
            cur = plsc.load_gather(q_buf, [gidx])
            prv = plsc.load_gather(q_buf, [jnp.maximum(gidx - 1, 0)])
            msk = jnp.where((cur != prv) | (gidx == 0), 1, 0).astype(jnp.int32)
            pc = plsc.cumsum(msk) + pcarry
            pos = pc - 1
            plsc.store_scatter(out_buf, [pos], cur)
            pcarry = jnp.max(pc)

        pltpu.sync_copy(out_buf, out_hbm.at[f])


def kernel(X, max_bin):
    del max_bin
    ranks_np, frac_np = _rank_tables()
    ranks = jnp.asarray(ranks_np)
    frac = jnp.asarray(frac_np)

    mesh = plsc.VectorSubcoreMesh(core_axis_name="c", subcore_axis_name="s")

    hist, mm = pl.kernel(
        _hist_body,
        out_type=(
            jax.ShapeDtypeStruct((NC * NS * NB,), jnp.int32),
            jax.ShapeDtypeStruct((NC * 2 * 16,), jnp.float32),
        ),
        mesh=mesh,
        scratch_types=[
            pltpu.VMEM((CHUNK, 16), jnp.float32),
            pltpu.VMEM((CHUNK, 16), jnp.float32),
            pltpu.VMEM((CHUNK * 16,), jnp.int32),
            pltpu.VMEM((CHUNK * 16,), jnp.int32),
            pltpu.VMEM((CHUNK * 16,), jnp.int32),
            pltpu.VMEM((32,), jnp.float32),
            pltpu.VMEM((NS * 32,), jnp.float32),
            pltpu.VMEM_SHARED((NS * NB,), jnp.int32),
            pltpu.VMEM_SHARED((NS * 32,), jnp.float32),
            pltpu.SemaphoreType.DMA,
            pltpu.SemaphoreType.DMA,
            pltpu.SemaphoreType.DMA,
            pltpu.SemaphoreType.DMA,
        ],
        compiler_params=pltpu.CompilerParams(needs_layout_passes=False,
                                             use_tc_tiling_on_sc=True),
        name="pgbm_hist",
    )(X, ranks)

    bins = pl.kernel(
        _quant_body,
        out_type=jax.ShapeDtypeStruct((N_FEAT, NQ), jnp.float32),
        mesh=mesh,
        scratch_types=[
            pltpu.VMEM((NB,), jnp.int32),
            pltpu.VMEM((BLK,), jnp.int32),
            pltpu.VMEM((BLK,), jnp.int32),
            pltpu.VMEM((2 * NQ,), jnp.int32),
            pltpu.VMEM((NQ,), jnp.float32),
            pltpu.VMEM((2 * NQ,), jnp.float32),
            pltpu.VMEM((NQ,), jnp.float32),
            pltpu.VMEM((NQ,), jnp.float32),
            pltpu.VMEM((64,), jnp.float32),
        ],
        compiler_params=pltpu.CompilerParams(needs_layout_passes=False, use_tc_tiling_on_sc=False),
        name="pgbm_quant",
    )(hist, mm, ranks, frac)

    return bins


# speedup vs baseline: 1.0294x; 1.0294x over previous
import jax
import jax.numpy as jnp
import numpy as np
from jax import lax
from jax.experimental import pallas as pl
from jax.experimental.pallas import tpu as pltpu
from jax.experimental.pallas import tpu_sc as plsc

N_ROWS = 2097152
N_FEAT = 16
LOG2NB = 15
NB = 1 << LOG2NB
SHIFT = 32 - LOG2NB
HALF = NB // 2
NQ = 256
NC = 2
NS = 16
NW = NC * NS
ROWS_PER_W = N_ROWS // NW
CHUNK = 256
NCHUNKS = ROWS_PER_W // CHUNK
BLK = 16384
NBLK = NB // BLK

_MASK31 = np.int32(0x7FFFFFFF)


def _rank_tables():
    k = np.arange(NQ, dtype=np.float64)
    pos = k * (N_ROWS - 1) / (NQ - 1)
    rlo = np.floor(pos).astype(np.int64)
    frac = (pos - rlo).astype(np.float32)
    rhi = np.minimum(rlo + 1, N_ROWS - 1)
    ranks = np.empty((2 * NQ,), dtype=np.int32)
    ranks[0::2] = rlo
    ranks[1::2] = rhi
    return ranks, frac


def _hist_body(x_hbm, ranks_hbm, hist_hbm, mm_hbm,
               in0, in1, idx0, idx1, ones_buf, mm_buf, mmred_buf,
               hist_sh, mm_sh,
               sem_in0, sem_in1, sem_sc0, sem_sc1):
    del ranks_hbm
    c = lax.axis_index("c")
    s = lax.axis_index("s")
    feat_iota = lax.iota(jnp.int32, 16)
    zeros16 = jnp.zeros((16,), jnp.int32)
    ones16 = jnp.ones((16,), jnp.int32)
    ins = (in0, in1)
    idxs = (idx0, idx1)
    sems_in = (sem_in0, sem_in1)
    sems_sc = (sem_sc0, sem_sc1)

    def fill(i, _):
        idx0[pl.ds(i * 16, 16)] = zeros16
        ones_buf[pl.ds(i * 16, 16)] = ones16
        return 0
    lax.fori_loop(0, CHUNK, fill, 0)

    for j in range(NB // (CHUNK * 16)):
        pltpu.sync_copy(idx0,
                        hist_sh.at[pl.ds(s * NB + j * CHUNK * 16, CHUNK * 16)])
    plsc.subcore_barrier()

    wid = c * NS + s
    row_base = wid * ROWS_PER_W

    pltpu.async_copy(x_hbm.at[pl.ds(row_base, CHUNK)], in0, sem_in0)

    def pair_body(p, carry):
        vmin, vmax = carry
        for par in range(2):
            g = p * 2 + par
            buf = ins[par]
            idx = idxs[par]

            nxt = g + 1

            @pl.when(nxt < NCHUNKS)
            def _():
                pltpu.async_copy(
                    x_hbm.at[pl.ds(row_base + nxt * CHUNK, CHUNK)],
                    ins[1 - par], sems_in[1 - par])

            pltpu.make_async_copy(
                x_hbm.at[pl.ds(row_base, CHUNK)], buf, sems_in[par]).wait()

            @pl.when(p > 0)
            def _():
                pltpu.make_async_copy(
                    ones_buf, hist_sh.at[idx], sems_sc[par]).wait()

            def elem_body(i, mc):
                vmn, vmx = mc
                v = buf[i]
                u = plsc.bitcast(v, jnp.int32)
                k = jnp.where(u >= 0, u, u ^ _MASK31)
                b = (k >> SHIFT) + HALF
                idx[pl.ds(i * 16, 16)] = b + feat_iota * NB
                return jnp.minimum(vmn, v), jnp.maximum(vmx, v)

            vmin, vmax = lax.fori_loop(0, CHUNK, elem_body, (vmin, vmax))
            pltpu.async_copy(ones_buf, hist_sh.at[idx], sems_sc[par], add=True)
        return vmin, vmax

    inf16 = jnp.full((16,), jnp.inf, jnp.float32)
    vmin, vmax = lax.fori_loop(0, NCHUNKS // 2, pair_body, (inf16, -inf16))
    for par in range(2):
        pltpu.make_async_copy(ones_buf, hist_sh.at[idxs[par]],
                              sems_sc[par]).wait()

    mm_buf[pl.ds(0, 16)] = vmin
    mm_buf[pl.ds(16, 16)] = vmax
    pltpu.sync_copy(mm_buf, mm_sh.at[pl.ds(s * 32, 32)])
    plsc.subcore_barrier()

    @pl.when(s == 0)
    def _():
        pltpu.sync_copy(mm_sh, mmred_buf)

        def red(i, mc):
            vmn, vmx = mc
            return (jnp.minimum(vmn, mmred_buf[pl.ds(i * 32, 16)]),
                    jnp.maximum(vmx, mmred_buf[pl.ds(i * 32 + 16, 16)]))
        rmin, rmax = lax.fori_loop(0, NS, red, (inf16, -inf16))
        mm_buf[pl.ds(0, 16)] = rmin
        mm_buf[pl.ds(16, 16)] = rmax
        pltpu.sync_copy(mm_buf, mm_hbm.at[pl.ds(c * 32, 32)])

    for j in range(NB // (CHUNK * 16)):
        pltpu.sync_copy(
            hist_sh.at[pl.ds(s * NB + j * CHUNK * 16, CHUNK * 16)], idx0)
        pltpu.sync_copy(
            idx0,
            hist_hbm.at[pl.ds((c * NS + s) * NB + j * CHUNK * 16,
                              CHUNK * 16)])


def _quant_body(hist_hbm, mm_hbm, ranks_hbm, frac_hbm, out_hbm,
                cum_buf, bufa, bufb, ranks_buf, frac_buf, est_buf,
                q_buf, out_buf, mm_buf):
    c = lax.axis_index("c")
    s = lax.axis_index("s")

    @pl.when(s < 8)
    def _():
        f = c * 8 + s
        iota16 = lax.iota(jnp.int32, 16)

        pltpu.sync_copy(ranks_hbm, ranks_buf)
        pltpu.sync_copy(frac_hbm, frac_buf)
        pltpu.sync_copy(mm_hbm, mm_buf)

        def blk(j, carry):
            pltpu.sync_copy(hist_hbm.at[pl.ds(f * NB + j * BLK, BLK)], bufa)
            pltpu.sync_copy(hist_hbm.at[pl.ds((NS + f) * NB + j * BLK, BLK)],
                            bufb)

            def inner(i, cy):
                va = bufa[pl.ds(i * 16, 16)]
                vb = bufb[pl.ds(i * 16, 16)]
                cs = plsc.cumsum(va + vb) + cy
                cum_buf[pl.ds(j * BLK + i * 16, 16)] = cs
                return jnp.max(cs)
            return lax.fori_loop(0, BLK // 16, inner, carry)

        lax.fori_loop(0, NBLK, blk, jnp.int32(0))

        vmn = jnp.minimum(mm_buf[pl.ds(0, 16)], mm_buf[pl.ds(32, 16)])
        vmx = jnp.maximum(mm_buf[pl.ds(16, 16)], mm_buf[pl.ds(48, 16)])
        minf = jnp.min(jnp.where(iota16 == f, vmn, jnp.inf))
        maxf = jnp.max(jnp.where(iota16 == f, vmx, -jnp.inf))

        for v in range(32):
            r = ranks_buf[pl.ds(v * 16, 16)]
            t = r + 1
            b = jnp.zeros((16,), jnp.int32)
            step = NB // 2
            while step >= 1:
                probe = b + (step - 1)
                val = plsc.load_gather(cum_buf, [probe])
                b = jnp.where(val < t, b + step, b)
                step //= 2
            cb = plsc.load_gather(cum_buf, [b])
            prev = plsc.load_gather(cum_buf, [jnp.maximum(b - 1, 0)])
            cbef = jnp.where(b > 0, prev, 0)
            m = cb - cbef
            j = r - cbef
            fracb = (j.astype(jnp.float32) + 0.5) / m.astype(jnp.float32)
            keylo = (b - HALF) << SHIFT
            keyhi = keylo + ((1 << SHIFT) - 1)
            ulo = jnp.where(keylo >= 0, keylo, keylo ^ _MASK31)
            uhi = jnp.where(keyhi >= 0, keyhi, keyhi ^ _MASK31)
            xlo = plsc.bitcast(ulo, jnp.float32)
            xhi = plsc.bitcast(uhi, jnp.float32)
            est = xlo + fracb * (xhi - xlo)
            est_buf[pl.ds(v * 16, 16)] = est

        for v in range(16):
            idx_even = 32 * v + 2 * iota16
            e_lo = plsc.load_gather(est_buf, [idx_even])
            e_hi = plsc.load_gather(est_buf, [idx_even + 1])
            fr = frac_buf[pl.ds(v * 16, 16)]
            q = e_lo + fr * (e_hi - e_lo)
            if v == 0:
                q = jnp.where(iota16 == 0, minf, q)
            if v == 15:
                q = jnp.where(iota16 == 15, maxf, q)
            q_buf[pl.ds(v * 16, 16)] = q

        for v in range(16):
            out_buf[pl.ds(v * 16, 16)] = jnp.zeros((16,), jnp.float32) + maxf

        pcarry = jnp.int32(0)
        for v in range(16):
            gidx = v * 16 + iota16
            cur = plsc.load_gather(q_buf, [gidx])
            prv = plsc.load_gather(q_buf, [jnp.maximum(gidx - 1, 0)])
            msk = jnp.where((cur != prv) | (gidx == 0), 1, 0).astype(jnp.int32)
            pc = plsc.cumsum(msk) + pcarry
            pos = pc - 1
            plsc.store_scatter(out_buf, [pos], cur)
            pcarry = jnp.max(pc)

        pltpu.sync_copy(out_buf, out_hbm.at[f])


def kernel(X, max_bin):
    del max_bin
    ranks_np, frac_np = _rank_tables()
    ranks = jnp.asarray(ranks_np)
    frac = jnp.asarray(frac_np)

    mesh = plsc.VectorSubcoreMesh(core_axis_name="c", subcore_axis_name="s")

    hist, mm = pl.kernel(
        _hist_body,
        out_type=(
            jax.ShapeDtypeStruct((NC * NS * NB,), jnp.int32),
            jax.ShapeDtypeStruct((NC * 2 * 16,), jnp.float32),
        ),
        mesh=mesh,
        scratch_types=[
            pltpu.VMEM((CHUNK, 16), jnp.float32),
            pltpu.VMEM((CHUNK, 16), jnp.float32),
            pltpu.VMEM((CHUNK * 16,), jnp.int32),
            pltpu.VMEM((CHUNK * 16,), jnp.int32),
            pltpu.VMEM((CHUNK * 16,), jnp.int32),
            pltpu.VMEM((32,), jnp.float32),
            pltpu.VMEM((NS * 32,), jnp.float32),
            pltpu.VMEM_SHARED((NS * NB,), jnp.int32),
            pltpu.VMEM_SHARED((NS * 32,), jnp.float32),
            pltpu.SemaphoreType.DMA,
            pltpu.SemaphoreType.DMA,
            pltpu.SemaphoreType.DMA,
            pltpu.SemaphoreType.DMA,
        ],
        compiler_params=pltpu.CompilerParams(needs_layout_passes=False,
                                             use_tc_tiling_on_sc=True),
        name="pgbm_hist",
    )(X, ranks)

    bins = pl.kernel(
        _quant_body,
        out_type=jax.ShapeDtypeStruct((N_FEAT, NQ), jnp.float32),
        mesh=mesh,
        scratch_types=[
            pltpu.VMEM((NB,), jnp.int32),
            pltpu.VMEM((BLK,), jnp.int32),
            pltpu.VMEM((BLK,), jnp.int32),
            pltpu.VMEM((2 * NQ,), jnp.int32),
            pltpu.VMEM((NQ,), jnp.float32),
            pltpu.VMEM((2 * NQ,), jnp.float32),
            pltpu.VMEM((NQ,), jnp.float32),
            pltpu.VMEM((NQ,), jnp.float32),
            pltpu.VMEM((64,), jnp.float32),
        ],
        compiler_params=pltpu.CompilerParams(needs_layout_passes=False, use_tc_tiling_on_sc=False),
        name="pgbm_quant",
    )(hist, mm, ranks, frac)

    return bins


# speedup vs baseline: 1.0323x; 1.0028x over previous
import jax
import jax.numpy as jnp
import numpy as np
from jax import lax
from jax.experimental import pallas as pl
from jax.experimental.pallas import tpu as pltpu
from jax.experimental.pallas import tpu_sc as plsc

N_ROWS = 2097152
N_FEAT = 16
LOG2NB = 15
NB = 1 << LOG2NB
SHIFT = 32 - LOG2NB
HALF = NB // 2
NQ = 256
NC = 2
NS = 16
NW = NC * NS
ROWS_PER_W = N_ROWS // NW
CHUNK = 256
NCHUNKS = ROWS_PER_W // CHUNK
BLK = 16384
NBLK = NB // BLK

_MASK31 = np.int32(0x7FFFFFFF)


def _rank_tables():
    k = np.arange(NQ, dtype=np.float64)
    pos = k * (N_ROWS - 1) / (NQ - 1)
    rlo = np.floor(pos).astype(np.int64)
    frac = (pos - rlo).astype(np.float32)
    rhi = np.minimum(rlo + 1, N_ROWS - 1)
    ranks = np.empty((2 * NQ,), dtype=np.int32)
    ranks[0::2] = rlo
    ranks[1::2] = rhi
    return ranks, frac


def _hist_body(x_hbm, ranks_hbm, hist_hbm, mm_hbm,
               in0, in1, idx0, idx1, ones_buf, mm_buf, mmred_buf,
               hist_sh, mm_sh,
               sem_in0, sem_in1, sem_sc0, sem_sc1):
    del ranks_hbm
    c = lax.axis_index("c")
    s = lax.axis_index("s")
    feat_iota = lax.iota(jnp.int32, 16)
    zeros16 = jnp.zeros((16,), jnp.int32)
    ones16 = jnp.ones((16,), jnp.int32)
    ins = (in0, in1)
    idxs = (idx0, idx1)
    sems_in = (sem_in0, sem_in1)
    sems_sc = (sem_sc0, sem_sc1)

    def fill(i, _):
        idx0[pl.ds(i * 16, 16)] = zeros16
        ones_buf[pl.ds(i * 16, 16)] = ones16
        return 0
    lax.fori_loop(0, 2 * CHUNK, fill, 0)

    for j in range(NB // (2 * CHUNK * 16)):
        pltpu.sync_copy(
            idx0, hist_sh.at[pl.ds(s * NB + j * 2 * CHUNK * 16,
                                   2 * CHUNK * 16)])
    plsc.subcore_barrier()

    wid = c * NS + s
    row_base = wid * ROWS_PER_W

    pltpu.async_copy(x_hbm.at[pl.ds(row_base, CHUNK)], in0, sem_in0)

    def quad_body(p, carry):
        vmin, vmax = carry
        for q in range(2):
            idx = idxs[q]

            @pl.when(p > 0)
            def _():
                pltpu.make_async_copy(
                    ones_buf, hist_sh.at[idx], sems_sc[q]).wait()

            for par in range(2):
                g = p * 4 + q * 2 + par
                buf = ins[par]
                nxt = g + 1

                @pl.when(nxt < NCHUNKS)
                def _():
                    pltpu.async_copy(
                        x_hbm.at[pl.ds(row_base + nxt * CHUNK, CHUNK)],
                        ins[1 - par], sems_in[1 - par])

                pltpu.make_async_copy(
                    x_hbm.at[pl.ds(row_base, CHUNK)], buf, sems_in[par]).wait()

                def elem_body(i, mc, buf=buf, idx=idx, par=par):
                    vmn, vmx = mc
                    v = buf[i]
                    u = plsc.bitcast(v, jnp.int32)
                    k = jnp.where(u >= 0, u, u ^ _MASK31)
                    b = (k >> SHIFT) + HALF
                    idx[pl.ds((par * CHUNK + i) * 16, 16)] = b + feat_iota * NB
                    return jnp.minimum(vmn, v), jnp.maximum(vmx, v)

                vmin, vmax = lax.fori_loop(0, CHUNK, elem_body, (vmin, vmax))
            pltpu.async_copy(ones_buf, hist_sh.at[idx], sems_sc[q], add=True)
        return vmin, vmax

    inf16 = jnp.full((16,), jnp.inf, jnp.float32)
    vmin, vmax = lax.fori_loop(0, NCHUNKS // 4, quad_body, (inf16, -inf16))
    for q in range(2):
        pltpu.make_async_copy(ones_buf, hist_sh.at[idxs[q]],
                              sems_sc[q]).wait()

    mm_buf[pl.ds(0, 16)] = vmin
    mm_buf[pl.ds(16, 16)] = vmax
    pltpu.sync_copy(mm_buf, mm_sh.at[pl.ds(s * 32, 32)])
    plsc.subcore_barrier()

    @pl.when(s == 0)
    def _():
        pltpu.sync_copy(mm_sh, mmred_buf)

        def red(i, mc):
            vmn, vmx = mc
            return (jnp.minimum(vmn, mmred_buf[pl.ds(i * 32, 16)]),
                    jnp.maximum(vmx, mmred_buf[pl.ds(i * 32 + 16, 16)]))
        rmin, rmax = lax.fori_loop(0, NS, red, (inf16, -inf16))
        mm_buf[pl.ds(0, 16)] = rmin
        mm_buf[pl.ds(16, 16)] = rmax
        pltpu.sync_copy(mm_buf, mm_hbm.at[pl.ds(c * 32, 32)])

    for j in range(NB // (2 * CHUNK * 16)):
        pltpu.sync_copy(
            hist_sh.at[pl.ds(s * NB + j * 2 * CHUNK * 16, 2 * CHUNK * 16)],
            idx0)
        pltpu.sync_copy(
            idx0,
            hist_hbm.at[pl.ds((c * NS + s) * NB + j * 2 * CHUNK * 16,
                              2 * CHUNK * 16)])


def _quant_body(hist_hbm, mm_hbm, ranks_hbm, frac_hbm, out_hbm,
                cum_buf, bufa, bufb, ranks_buf, frac_buf, est_buf,
                q_buf, out_buf, mm_buf):
    c = lax.axis_index("c")
    s = lax.axis_index("s")

    @pl.when(s < 8)
    def _():
        f = c * 8 + s
        iota16 = lax.iota(jnp.int32, 16)

        pltpu.sync_copy(ranks_hbm, ranks_buf)
        pltpu.sync_copy(frac_hbm, frac_buf)
        pltpu.sync_copy(mm_hbm, mm_buf)

        def blk(j, carry):
            pltpu.sync_copy(hist_hbm.at[pl.ds(f * NB + j * BLK, BLK)], bufa)
            pltpu.sync_copy(hist_hbm.at[pl.ds((NS + f) * NB + j * BLK, BLK)],
                            bufb)

            def inner(i, cy):
                va = bufa[pl.ds(i * 16, 16)]
                vb = bufb[pl.ds(i * 16, 16)]
                cs = plsc.cumsum(va + vb) + cy
                cum_buf[pl.ds(j * BLK + i * 16, 16)] = cs
                return jnp.max(cs)
            return lax.fori_loop(0, BLK // 16, inner, carry)

        lax.fori_loop(0, NBLK, blk, jnp.int32(0))

        vmn = jnp.minimum(mm_buf[pl.ds(0, 16)], mm_buf[pl.ds(32, 16)])
        vmx = jnp.maximum(mm_buf[pl.ds(16, 16)], mm_buf[pl.ds(48, 16)])
        minf = jnp.min(jnp.where(iota16 == f, vmn, jnp.inf))
        maxf = jnp.max(jnp.where(iota16 == f, vmx, -jnp.inf))

        for v in range(32):
            r = ranks_buf[pl.ds(v * 16, 16)]
            t = r + 1
            b = jnp.zeros((16,), jnp.int32)
            step = NB // 2
            while step >= 1:
                probe = b + (step - 1)
                val = plsc.load_gather(cum_buf, [probe])
                b = jnp.where(val < t, b + step, b)
                step //= 2
            cb = plsc.load_gather(cum_buf, [b])
            prev = plsc.load_gather(cum_buf, [jnp.maximum(b - 1, 0)])
            cbef = jnp.where(b > 0, prev, 0)
            m = cb - cbef
            j = r - cbef
            fracb = (j.astype(jnp.float32) + 0.5) / m.astype(jnp.float32)
            keylo = (b - HALF) << SHIFT
            keyhi = keylo + ((1 << SHIFT) - 1)
            ulo = jnp.where(keylo >= 0, keylo, keylo ^ _MASK31)
            uhi = jnp.where(keyhi >= 0, keyhi, keyhi ^ _MASK31)
            xlo = plsc.bitcast(ulo, jnp.float32)
            xhi = plsc.bitcast(uhi, jnp.float32)
            est = xlo + fracb * (xhi - xlo)
            est_buf[pl.ds(v * 16, 16)] = est

        for v in range(16):
            idx_even = 32 * v + 2 * iota16
            e_lo = plsc.load_gather(est_buf, [idx_even])
            e_hi = plsc.load_gather(est_buf, [idx_even + 1])
            fr = frac_buf[pl.ds(v * 16, 16)]
            q = e_lo + fr * (e_hi - e_lo)
            if v == 0:
                q = jnp.where(iota16 == 0, minf, q)
            if v == 15:
                q = jnp.where(iota16 == 15, maxf, q)
            q_buf[pl.ds(v * 16, 16)] = q

        for v in range(16):
            out_buf[pl.ds(v * 16, 16)] = jnp.zeros((16,), jnp.float32) + maxf

        pcarry = jnp.int32(0)
        for v in range(16):
            gidx = v * 16 + iota16
            cur = plsc.load_gather(q_buf, [gidx])
            prv = plsc.load_gather(q_buf, [jnp.maximum(gidx - 1, 0)])
            msk = jnp.where((cur != prv) | (gidx == 0), 1, 0).astype(jnp.int32)
            pc = plsc.cumsum(msk) + pcarry
            pos = pc - 1
            plsc.store_scatter(out_buf, [pos], cur)
            pcarry = jnp.max(pc)

        pltpu.sync_copy(out_buf, out_hbm.at[f])


def kernel(X, max_bin):
    del max_bin
    ranks_np, frac_np = _rank_tables()
    ranks = jnp.asarray(ranks_np)
    frac = jnp.asarray(frac_np)

    mesh = plsc.VectorSubcoreMesh(core_axis_name="c", subcore_axis_name="s")

    hist, mm = pl.kernel(
        _hist_body,
        out_type=(
            jax.ShapeDtypeStruct((NC * NS * NB,), jnp.int32),
            jax.ShapeDtypeStruct((NC * 2 * 16,), jnp.float32),
        ),
        mesh=mesh,
        scratch_types=[
            pltpu.VMEM((CHUNK, 16), jnp.float32),
            pltpu.VMEM((CHUNK, 16), jnp.float32),
            pltpu.VMEM((2 * CHUNK * 16,), jnp.int32),
            pltpu.VMEM((2 * CHUNK * 16,), jnp.int32),
            pltpu.VMEM((2 * CHUNK * 16,), jnp.int32),
            pltpu.VMEM((32,), jnp.float32),
            pltpu.VMEM((NS * 32,), jnp.float32),
            pltpu.VMEM_SHARED((NS * NB,), jnp.int32),
            pltpu.VMEM_SHARED((NS * 32,), jnp.float32),
            pltpu.SemaphoreType.DMA,
            pltpu.SemaphoreType.DMA,
            pltpu.SemaphoreType.DMA,
            pltpu.SemaphoreType.DMA,
        ],
        compiler_params=pltpu.CompilerParams(needs_layout_passes=False,
                                             use_tc_tiling_on_sc=True),
        name="pgbm_hist",
    )(X, ranks)

    bins = pl.kernel(
        _quant_body,
        out_type=jax.ShapeDtypeStruct((N_FEAT, NQ), jnp.float32),
        mesh=mesh,
        scratch_types=[
            pltpu.VMEM((NB,), jnp.int32),
            pltpu.VMEM((BLK,), jnp.int32),
            pltpu.VMEM((BLK,), jnp.int32),
            pltpu.VMEM((2 * NQ,), jnp.int32),
            pltpu.VMEM((NQ,), jnp.float32),
            pltpu.VMEM((2 * NQ,), jnp.float32),
            pltpu.VMEM((NQ,), jnp.float32),
            pltpu.VMEM((NQ,), jnp.float32),
            pltpu.VMEM((64,), jnp.float32),
        ],
        compiler_params=pltpu.CompilerParams(needs_layout_passes=False, use_tc_tiling_on_sc=False),
        name="pgbm_quant",
    )(hist, mm, ranks, frac)

    return bins


# speedup vs baseline: 1.1010x; 1.0666x over previous
import jax
import jax.numpy as jnp
import numpy as np
from jax import lax
from jax.experimental import pallas as pl
from jax.experimental.pallas import tpu as pltpu
from jax.experimental.pallas import tpu_sc as plsc

N_ROWS = 2097152
N_FEAT = 16
LOG2NB = 15
NB = 1 << LOG2NB
SHIFT = 32 - LOG2NB
HALF = NB // 2
NQ = 256
NC = 2
NS = 16
NW = NC * NS
ROWS_PER_W = N_ROWS // NW
CHUNK = 128
NCHUNKS = ROWS_PER_W // CHUNK
BLK = 16384
NBLK = NB // BLK

_MASK31 = np.int32(0x7FFFFFFF)


def _rank_tables():
    k = np.arange(NQ, dtype=np.float64)
    pos = k * (N_ROWS - 1) / (NQ - 1)
    rlo = np.floor(pos).astype(np.int64)
    frac = (pos - rlo).astype(np.float32)
    rhi = np.minimum(rlo + 1, N_ROWS - 1)
    ranks = np.empty((2 * NQ,), dtype=np.int32)
    ranks[0::2] = rlo
    ranks[1::2] = rhi
    return ranks, frac


def _hist_body(x_hbm, ranks_hbm, hist_hbm, mm_hbm,
               in0, in1, in2, in3, idx0, idx1, ones_buf, mm_buf, mmred_buf,
               hist_sh, mm_sh,
               sem_in0, sem_in1, sem_in2, sem_in3, sem_sc0, sem_sc1):
    del ranks_hbm
    c = lax.axis_index("c")
    s = lax.axis_index("s")
    feat_iota = lax.iota(jnp.int32, 16)
    zeros16 = jnp.zeros((16,), jnp.int32)
    ones16 = jnp.ones((16,), jnp.int32)
    ins = (in0, in1, in2, in3)
    idxs = (idx0, idx1)
    sems_in = (sem_in0, sem_in1, sem_in2, sem_in3)
    sems_sc = (sem_sc0, sem_sc1)

    def fill(i, _):
        idx0[pl.ds(i * 16, 16)] = zeros16
        ones_buf[pl.ds(i * 16, 16)] = ones16
        return 0
    lax.fori_loop(0, 2 * CHUNK, fill, 0)

    for j in range(NB // (2 * CHUNK * 16)):
        pltpu.sync_copy(
            idx0, hist_sh.at[pl.ds(s * NB + j * 2 * CHUNK * 16,
                                   2 * CHUNK * 16)])
    plsc.subcore_barrier()

    wid = c * NS + s
    row_base = wid * ROWS_PER_W

    for w in range(3):
        pltpu.async_copy(x_hbm.at[pl.ds(row_base + w * CHUNK, CHUNK)],
                         ins[w], sems_in[w])

    def quad_body(p, carry):
        vmin, vmax = carry
        for par in range(4):
            g = p * 4 + par
            q = par // 2
            buf = ins[par]
            idx = idxs[q]

            if par % 2 == 0:

                @pl.when(p > 0)
                def _():
                    pltpu.make_async_copy(
                        ones_buf, hist_sh.at[idx], sems_sc[q]).wait()

            nxt = g + 3

            @pl.when(nxt < NCHUNKS)
            def _():
                pltpu.async_copy(
                    x_hbm.at[pl.ds(row_base + nxt * CHUNK, CHUNK)],
                    ins[(par + 3) % 4], sems_in[(par + 3) % 4])

            pltpu.make_async_copy(
                x_hbm.at[pl.ds(row_base, CHUNK)], buf, sems_in[par]).wait()

            def elem_body(i, mc, buf=buf, idx=idx, par=par):
                vmn, vmx = mc
                v = buf[i]
                u = plsc.bitcast(v, jnp.int32)
                k = jnp.where(u >= 0, u, u ^ _MASK31)
                b = (k >> SHIFT) + HALF
                idx[pl.ds(((par % 2) * CHUNK + i) * 16, 16)] = (
                    b + feat_iota * NB)
                return jnp.minimum(vmn, v), jnp.maximum(vmx, v)

            vmin, vmax = lax.fori_loop(0, CHUNK, elem_body, (vmin, vmax))
            if par % 2 == 1:
                pltpu.async_copy(ones_buf, hist_sh.at[idx], sems_sc[q],
                                 add=True)
        return vmin, vmax

    inf16 = jnp.full((16,), jnp.inf, jnp.float32)
    vmin, vmax = lax.fori_loop(0, NCHUNKS // 4, quad_body, (inf16, -inf16))
    for q in range(2):
        pltpu.make_async_copy(ones_buf, hist_sh.at[idxs[q]],
                              sems_sc[q]).wait()

    mm_buf[pl.ds(0, 16)] = vmin
    mm_buf[pl.ds(16, 16)] = vmax
    pltpu.sync_copy(mm_buf, mm_sh.at[pl.ds(s * 32, 32)])
    plsc.subcore_barrier()

    @pl.when(s == 0)
    def _():
        pltpu.sync_copy(mm_sh, mmred_buf)

        def red(i, mc):
            vmn, vmx = mc
            return (jnp.minimum(vmn, mmred_buf[pl.ds(i * 32, 16)]),
                    jnp.maximum(vmx, mmred_buf[pl.ds(i * 32 + 16, 16)]))
        rmin, rmax = lax.fori_loop(0, NS, red, (inf16, -inf16))
        mm_buf[pl.ds(0, 16)] = rmin
        mm_buf[pl.ds(16, 16)] = rmax
        pltpu.sync_copy(mm_buf, mm_hbm.at[pl.ds(c * 32, 32)])

    for j in range(NB // (2 * CHUNK * 16)):
        pltpu.sync_copy(
            hist_sh.at[pl.ds(s * NB + j * 2 * CHUNK * 16, 2 * CHUNK * 16)],
            idx0)
        pltpu.sync_copy(
            idx0,
            hist_hbm.at[pl.ds((c * NS + s) * NB + j * 2 * CHUNK * 16,
                              2 * CHUNK * 16)])


def _quant_body(hist_hbm, mm_hbm, ranks_hbm, frac_hbm, out_hbm,
                cum_buf, bufa, bufb, ranks_buf, frac_buf, est_buf,
                q_buf, out_buf, mm_buf):
    c = lax.axis_index("c")
    s = lax.axis_index("s")

    @pl.when(s < 8)
    def _():
        f = c * 8 + s
        iota16 = lax.iota(jnp.int32, 16)

        pltpu.sync_copy(ranks_hbm, ranks_buf)
        pltpu.sync_copy(frac_hbm, frac_buf)
        pltpu.sync_copy(mm_hbm, mm_buf)

        def blk(j, carry):
            pltpu.sync_copy(hist_hbm.at[pl.ds(f * NB + j * BLK, BLK)], bufa)
            pltpu.sync_copy(hist_hbm.at[pl.ds((NS + f) * NB + j * BLK, BLK)],
                            bufb)

            def inner(i, cy):
                va = bufa[pl.ds(i * 16, 16)]
                vb = bufb[pl.ds(i * 16, 16)]
                cs = plsc.cumsum(va + vb) + cy
                cum_buf[pl.ds(j * BLK + i * 16, 16)] = cs
                return jnp.max(cs)
            return lax.fori_loop(0, BLK // 16, inner, carry)

        lax.fori_loop(0, NBLK, blk, jnp.int32(0))

        vmn = jnp.minimum(mm_buf[pl.ds(0, 16)], mm_buf[pl.ds(32, 16)])
        vmx = jnp.maximum(mm_buf[pl.ds(16, 16)], mm_buf[pl.ds(48, 16)])
        minf = jnp.min(jnp.where(iota16 == f, vmn, jnp.inf))
        maxf = jnp.max(jnp.where(iota16 == f, vmx, -jnp.inf))

        for v in range(32):
            r = ranks_buf[pl.ds(v * 16, 16)]
            t = r + 1
            b = jnp.zeros((16,), jnp.int32)
            step = NB // 2
            while step >= 1:
                probe = b + (step - 1)
                val = plsc.load_gather(cum_buf, [probe])
                b = jnp.where(val < t, b + step, b)
                step //= 2
            cb = plsc.load_gather(cum_buf, [b])
            prev = plsc.load_gather(cum_buf, [jnp.maximum(b - 1, 0)])
            cbef = jnp.where(b > 0, prev, 0)
            m = cb - cbef
            j = r - cbef
            fracb = (j.astype(jnp.float32) + 0.5) / m.astype(jnp.float32)
            keylo = (b - HALF) << SHIFT
            keyhi = keylo + ((1 << SHIFT) - 1)
            ulo = jnp.where(keylo >= 0, keylo, keylo ^ _MASK31)
            uhi = jnp.where(keyhi >= 0, keyhi, keyhi ^ _MASK31)
            xlo = plsc.bitcast(ulo, jnp.float32)
            xhi = plsc.bitcast(uhi, jnp.float32)
            est = xlo + fracb * (xhi - xlo)
            est_buf[pl.ds(v * 16, 16)] = est

        for v in range(16):
            idx_even = 32 * v + 2 * iota16
            e_lo = plsc.load_gather(est_buf, [idx_even])
            e_hi = plsc.load_gather(est_buf, [idx_even + 1])
            fr = frac_buf[pl.ds(v * 16, 16)]
            q = e_lo + fr * (e_hi - e_lo)
            if v == 0:
                q = jnp.where(iota16 == 0, minf, q)
            if v == 15:
                q = jnp.where(iota16 == 15, maxf, q)
            q_buf[pl.ds(v * 16, 16)] = q

        for v in range(16):
            out_buf[pl.ds(v * 16, 16)] = jnp.zeros((16,), jnp.float32) + maxf

        pcarry = jnp.int32(0)
        for v in range(16):
            gidx = v * 16 + iota16
            cur = plsc.load_gather(q_buf, [gidx])
            prv = plsc.load_gather(q_buf, [jnp.maximum(gidx - 1, 0)])
            msk = jnp.where((cur != prv) | (gidx == 0), 1, 0).astype(jnp.int32)
            pc = plsc.cumsum(msk) + pcarry
            pos = pc - 1
            plsc.store_scatter(out_buf, [pos], cur)
            pcarry = jnp.max(pc)

        pltpu.sync_copy(out_buf, out_hbm.at[f])


def kernel(X, max_bin):
    del max_bin
    ranks_np, frac_np = _rank_tables()
    ranks = jnp.asarray(ranks_np)
    frac = jnp.asarray(frac_np)

    mesh = plsc.VectorSubcoreMesh(core_axis_name="c", subcore_axis_name="s")

    hist, mm = pl.kernel(
        _hist_body,
        out_type=(
            jax.ShapeDtypeStruct((NC * NS * NB,), jnp.int32),
            jax.ShapeDtypeStruct((NC * 2 * 16,), jnp.float32),
        ),
        mesh=mesh,
        scratch_types=[
            pltpu.VMEM((CHUNK, 16), jnp.float32),
            pltpu.VMEM((CHUNK, 16), jnp.float32),
            pltpu.VMEM((CHUNK, 16), jnp.float32),
            pltpu.VMEM((CHUNK, 16), jnp.float32),
            pltpu.VMEM((2 * CHUNK * 16,), jnp.int32),
            pltpu.VMEM((2 * CHUNK * 16,), jnp.int32),
            pltpu.VMEM((2 * CHUNK * 16,), jnp.int32),
            pltpu.VMEM((32,), jnp.float32),
            pltpu.VMEM((NS * 32,), jnp.float32),
            pltpu.VMEM_SHARED((NS * NB,), jnp.int32),
            pltpu.VMEM_SHARED((NS * 32,), jnp.float32),
            pltpu.SemaphoreType.DMA,
            pltpu.SemaphoreType.DMA,
            pltpu.SemaphoreType.DMA,
            pltpu.SemaphoreType.DMA,
            pltpu.SemaphoreType.DMA,
            pltpu.SemaphoreType.DMA,
        ],
        compiler_params=pltpu.CompilerParams(needs_layout_passes=False,
                                             use_tc_tiling_on_sc=True),
        name="pgbm_hist",
    )(X, ranks)

    bins = pl.kernel(
        _quant_body,
        out_type=jax.ShapeDtypeStruct((N_FEAT, NQ), jnp.float32),
        mesh=mesh,
        scratch_types=[
            pltpu.VMEM((NB,), jnp.int32),
            pltpu.VMEM((BLK,), jnp.int32),
            pltpu.VMEM((BLK,), jnp.int32),
            pltpu.VMEM((2 * NQ,), jnp.int32),
            pltpu.VMEM((NQ,), jnp.float32),
            pltpu.VMEM((2 * NQ,), jnp.float32),
            pltpu.VMEM((NQ,), jnp.float32),
            pltpu.VMEM((NQ,), jnp.float32),
            pltpu.VMEM((64,), jnp.float32),
        ],
        compiler_params=pltpu.CompilerParams(needs_layout_passes=False, use_tc_tiling_on_sc=False),
        name="pgbm_quant",
    )(hist, mm, ranks, frac)

    return bins
